# baseline (speedup 1.0000x reference)
import functools

import jax
import jax.numpy as jnp
from jax import lax
from jax.experimental import pallas as pl
from jax.experimental.pallas import tpu as pltpu
from jax.experimental.pallas import tpu_sc as plsc


@functools.cache
def _make_gather(V, D, B):
    info = plsc.get_sparse_core_info()
    NC, NS = info.num_cores, info.num_subcores
    NW = NC * NS
    assert B % NW == 0
    b_per_w = B // NW
    CHUNK = 1024
    assert b_per_w % CHUNK == 0
    n_chunks = b_per_w // CHUNK
    mesh = plsc.VectorSubcoreMesh(core_axis_name="c", subcore_axis_name="s")

    @functools.partial(
        pl.kernel,
        mesh=mesh,
        out_type=jax.ShapeDtypeStruct((B, D), jnp.float32),
        scratch_types=[
            pltpu.VMEM((CHUNK,), jnp.int32),
            pltpu.VMEM((CHUNK, D), jnp.float32),
            pltpu.SemaphoreType.DMA,
        ],
        compiler_params=pltpu.CompilerParams(use_tc_tiling_on_sc=False),
    )
    def gather_kernel(idx_hbm, table_hbm, out_hbm, idx_v, rows_v, sem):
        wid = lax.axis_index("s") * NC + lax.axis_index("c")
        base = wid * b_per_w

        def body(c, carry):
            off = base + c * CHUNK
            pltpu.sync_copy(idx_hbm.at[pl.ds(off, CHUNK)], idx_v)
            pltpu.async_copy(table_hbm.at[idx_v], rows_v, sem).wait()
            pltpu.sync_copy(rows_v, out_hbm.at[pl.ds(off, CHUNK)])
            return carry

        lax.fori_loop(0, n_chunks, body, 0)

    return gather_kernel


@jax.jit
def kernel(x, weight):
    B, H = x.shape
    V, D = weight.shape
    flat = x.reshape(B * H).astype(jnp.int32)
    out = _make_gather(V, D, B * H)(flat, weight)
    return out.reshape(B, H, D)


# speedup vs baseline: 1.0158x; 1.0158x over previous
import functools

import jax
import jax.numpy as jnp
from jax import lax
from jax.experimental import pallas as pl
from jax.experimental.pallas import tpu as pltpu
from jax.experimental.pallas import tpu_sc as plsc


@functools.cache
def _make_gather(V, D, B):
    info = plsc.get_sparse_core_info()
    NC, NS = info.num_cores, info.num_subcores
    NW = NC * NS
    assert B % NW == 0
    b_per_w = B // NW
    CHUNK = 800
    assert b_per_w % (2 * CHUNK) == 0
    n_pairs = b_per_w // (2 * CHUNK)
    mesh = plsc.VectorSubcoreMesh(core_axis_name="c", subcore_axis_name="s")

    @functools.partial(
        pl.kernel,
        mesh=mesh,
        out_type=jax.ShapeDtypeStruct((B, D), jnp.float32),
        scratch_types=[
            pltpu.VMEM((b_per_w,), jnp.int32),
            pltpu.VMEM((CHUNK, D), jnp.float32),
            pltpu.VMEM((CHUNK, D), jnp.float32),
            pltpu.SemaphoreType.DMA,
            pltpu.SemaphoreType.DMA,
            pltpu.SemaphoreType.DMA,
            pltpu.SemaphoreType.DMA,
        ],
        compiler_params=pltpu.CompilerParams(use_tc_tiling_on_sc=False),
    )
    def gather_kernel(idx_hbm, table_hbm, out_hbm, idx_all, rows0, rows1,
                      gsem0, gsem1, osem0, osem1):
        wid = lax.axis_index("s") * NC + lax.axis_index("c")
        base = wid * b_per_w
        pltpu.sync_copy(idx_hbm.at[pl.ds(base, b_per_w)], idx_all)

        def g_start(c, buf, sem):
            return pltpu.async_copy(
                table_hbm.at[idx_all.at[pl.ds(c * CHUNK, CHUNK)]], buf, sem)

        def s_start(c, buf, sem):
            return pltpu.async_copy(
                buf, out_hbm.at[pl.ds(base + c * CHUNK, CHUNK)], sem)

        def g_wait(buf, sem):
            pltpu.make_async_copy(table_hbm.at[idx_all.at[pl.ds(0, CHUNK)]],
                                  buf, sem).wait()

        def s_wait(buf, sem):
            pltpu.make_async_copy(buf, out_hbm.at[pl.ds(base, CHUNK)],
                                  sem).wait()

        g_start(0, rows0, gsem0)

        def body(p, carry):
            c0 = 2 * p
            c1 = c0 + 1
            g_wait(rows0, gsem0)
            s_start(c0, rows0, osem0)

            @pl.when(p > 0)
            def _():
                s_wait(rows1, osem1)

            g_start(c1, rows1, gsem1)
            g_wait(rows1, gsem1)
            s_start(c1, rows1, osem1)

            @pl.when(p < n_pairs - 1)
            def _():
                s_wait(rows0, osem0)
                g_start(c0 + 2, rows0, gsem0)

            return carry

        lax.fori_loop(0, n_pairs, body, 0)
        s_wait(rows0, osem0)
        s_wait(rows1, osem1)

    return gather_kernel


@jax.jit
def kernel(x, weight):
    B, H = x.shape
    V, D = weight.shape
    flat = x.reshape(B * H).astype(jnp.int32)
    out = _make_gather(V, D, B * H)(flat, weight)
    return out.reshape(B, H, D)
